# baseline (speedup 1.0000x reference)
import functools

import jax
import jax.numpy as jnp
from jax import lax
from jax.experimental import pallas as pl
from jax.experimental.pallas import tpu as pltpu
from jax.experimental.pallas import tpu_sc as plsc

_BR = 1000


_DN = (((1,), (1,)), ((), ()))


def _dense(x, w, b):
    N, D = x.shape

    def body(x_ref, w_ref, b_ref, o_ref):
        o_ref[...] = (
            lax.dot_general(x_ref[...], w_ref[...], _DN,
                            preferred_element_type=jnp.float32)
            + b_ref[...]
        ) * 0.5

    return pl.pallas_call(
        body,
        grid=(N // _BR,),
        in_specs=[
            pl.BlockSpec((_BR, D), lambda i: (i, 0)),
            pl.BlockSpec((D, D), lambda i: (0, 0)),
            pl.BlockSpec((1, D), lambda i: (0, 0)),
        ],
        out_specs=pl.BlockSpec((_BR, D), lambda i: (i, 0)),
        out_shape=jax.ShapeDtypeStruct((N, D), jnp.float32),
    )(x, w, b)


def _combine_relu_dense(t, p, w, b):
    N, D = t.shape
    NC = p.shape[0]

    def body(t_ref, p_ref, w_ref, b_ref, o_ref):
        h = t_ref[...] + jnp.sum(p_ref[...], axis=0)
        h = jnp.maximum(h, 0.0)
        o_ref[...] = (
            lax.dot_general(h, w_ref[...], _DN,
                            preferred_element_type=jnp.float32)
            + b_ref[...]
        ) * 0.5

    return pl.pallas_call(
        body,
        grid=(N // _BR,),
        in_specs=[
            pl.BlockSpec((_BR, D), lambda i: (i, 0)),
            pl.BlockSpec((NC, _BR, D), lambda i: (0, i, 0)),
            pl.BlockSpec((D, D), lambda i: (0, 0)),
            pl.BlockSpec((1, D), lambda i: (0, 0)),
        ],
        out_specs=pl.BlockSpec((_BR, D), lambda i: (i, 0)),
        out_shape=jax.ShapeDtypeStruct((N, D), jnp.float32),
    )(t, p, w, b)


def _combine(t, p):
    N, D = t.shape
    NC = p.shape[0]

    def body(t_ref, p_ref, o_ref):
        o_ref[...] = t_ref[...] + jnp.sum(p_ref[...], axis=0)

    return pl.pallas_call(
        body,
        grid=(N // _BR,),
        in_specs=[
            pl.BlockSpec((_BR, D), lambda i: (i, 0)),
            pl.BlockSpec((NC, _BR, D), lambda i: (0, i, 0)),
        ],
        out_specs=pl.BlockSpec((_BR, D), lambda i: (i, 0)),
        out_shape=jax.ShapeDtypeStruct((N, D), jnp.float32),
    )(t, p)


def _sc_aggregate(t, rc3, zeros):
    N, D = t.shape
    _, NW, nch, CH = rc3.shape
    info = plsc.get_sparse_core_info()
    NC, NS = info.num_cores, info.num_subcores
    assert NW == NC * NS and N % NS == 0 and D % 16 == 0
    RPT = N // NS
    NB = 4
    SB = max(s for s in range(NB, 41, NB) if nch % s == 0)
    NSB = nch // SB
    mesh = plsc.VectorSubcoreMesh(core_axis_name="c", subcore_axis_name="s")

    @functools.partial(
        pl.kernel,
        out_type=jax.ShapeDtypeStruct((NC, NS, RPT, D), jnp.float32),
        mesh=mesh,
        scratch_types=[
            pltpu.VMEM((SB, CH), jnp.int32),
            pltpu.VMEM((SB, CH), jnp.int32),
        ]
        + [pltpu.VMEM((CH, D), jnp.float32) for _ in range(NB)]
        + [pltpu.VMEM_SHARED((N, D), jnp.float32)]
        + [pltpu.SemaphoreType.DMA] * (2 * NB),
    )
    def k(t_hbm, rc_hbm, z_hbm, out_hbm, rowb, colb, *rest):
        rows = rest[:NB]
        acc = rest[NB]
        sg = rest[NB + 1:NB + 1 + NB]
        ss = rest[NB + 1 + NB:]
        cid = lax.axis_index("c")
        sid = lax.axis_index("s")
        wid = sid * NC + cid

        pltpu.sync_copy(z_hbm, acc.at[pl.ds(sid * RPT, RPT)])
        plsc.subcore_barrier()

        def fire_g(j, b):
            pltpu.async_copy(t_hbm.at[rowb.at[j]], rows[b], sg[b])

        def wait_g(j, b):
            pltpu.make_async_copy(t_hbm.at[rowb.at[j]], rows[b], sg[b]).wait()

        def fire_s(j, b):
            pltpu.async_copy(rows[b], acc.at[colb.at[j]], ss[b], add=True)

        def drain_s(j, b):
            pltpu.make_async_copy(rows[b], acc.at[colb.at[j]], ss[b]).wait()

        def step(c, drain, fire):
            b = c % NB
            wait_g(c, b)
            fire_s(c, b)
            if drain:
                drain_s(c - 2, (c - 2) % NB)
            if fire:
                fire_g(c + 2, (c + 2) % NB)

        def sblock(s, c):
            pltpu.sync_copy(rc_hbm.at[0, wid, pl.ds(s * SB, SB)], rowb)
            pltpu.sync_copy(rc_hbm.at[1, wid, pl.ds(s * SB, SB)], colb)
            fire_g(0, 0)
            fire_g(1, 1)
            step(0, False, True)
            step(1, False, True)

            def body(i, c2):
                j = NB * i + 2
                for b in range(NB):
                    jb = j + b
                    wait_g(jb, (2 + b) % NB)
                    fire_s(jb, (2 + b) % NB)
                    drain_s(jb - 2, b)
                    fire_g(jb + 2, b)
                return c2

            lax.fori_loop(0, (SB - 4) // NB, body, 0)
            step(SB - 2, True, False)
            step(SB - 1, True, False)
            drain_s(SB - 2, (SB - 2) % NB)
            drain_s(SB - 1, (SB - 1) % NB)
            return c

        lax.fori_loop(0, NSB, sblock, 0)
        plsc.subcore_barrier()
        pltpu.sync_copy(acc.at[pl.ds(sid * RPT, RPT)], out_hbm.at[cid, sid])

    return k(t, rc3, zeros).reshape(NC, N, D)


def kernel(node_features, edge_index, weight0, bias0, weight1, bias1, hidden_dim):
    del hidden_dim
    N = node_features.shape[0]
    E = edge_index.shape[1]
    info = plsc.get_sparse_core_info()
    NW = info.num_cores * info.num_subcores
    EPW = E // NW
    assert E % NW == 0
    CH = max(c for c in range(1, 65) if EPW % c == 0 and (EPW // c) % 4 == 0)
    rc3 = edge_index.reshape(2, NW, EPW // CH, CH)
    NS = info.num_subcores
    zeros = jnp.zeros((N // NS, node_features.shape[1]), jnp.float32)
    t0 = _dense(node_features, weight0[0], bias0)
    p0 = _sc_aggregate(t0, rc3, zeros)
    t1 = _combine_relu_dense(t0, p0, weight1[0], bias1)
    p1 = _sc_aggregate(t1, rc3, zeros)
    return _combine(t1, p1)


# speedup vs baseline: 1.1836x; 1.1836x over previous
import functools

import jax
import jax.numpy as jnp
from jax import lax
from jax.experimental import pallas as pl
from jax.experimental.pallas import tpu as pltpu
from jax.experimental.pallas import tpu_sc as plsc

_BR = 1000


_DN = (((1,), (1,)), ((), ()))


def _dense(x, w, b):
    N, D = x.shape

    def body(x_ref, w_ref, b_ref, o_ref):
        o_ref[...] = (
            lax.dot_general(x_ref[...], w_ref[...], _DN,
                            preferred_element_type=jnp.float32)
            + b_ref[...]
        ) * 0.5

    return pl.pallas_call(
        body,
        grid=(N // _BR,),
        in_specs=[
            pl.BlockSpec((_BR, D), lambda i: (i, 0)),
            pl.BlockSpec((D, D), lambda i: (0, 0)),
            pl.BlockSpec((1, D), lambda i: (0, 0)),
        ],
        out_specs=pl.BlockSpec((_BR, D), lambda i: (i, 0)),
        out_shape=jax.ShapeDtypeStruct((N, D), jnp.float32),
    )(x, w, b)


def _combine_relu_dense(t, p, w, b):
    N, D = t.shape
    NC = p.shape[0]

    def body(t_ref, p_ref, w_ref, b_ref, o_ref):
        h = t_ref[...] + jnp.sum(p_ref[...], axis=0)
        h = jnp.maximum(h, 0.0)
        o_ref[...] = (
            lax.dot_general(h, w_ref[...], _DN,
                            preferred_element_type=jnp.float32)
            + b_ref[...]
        ) * 0.5

    return pl.pallas_call(
        body,
        grid=(N // _BR,),
        in_specs=[
            pl.BlockSpec((_BR, D), lambda i: (i, 0)),
            pl.BlockSpec((NC, _BR, D), lambda i: (0, i, 0)),
            pl.BlockSpec((D, D), lambda i: (0, 0)),
            pl.BlockSpec((1, D), lambda i: (0, 0)),
        ],
        out_specs=pl.BlockSpec((_BR, D), lambda i: (i, 0)),
        out_shape=jax.ShapeDtypeStruct((N, D), jnp.float32),
    )(t, p, w, b)


def _combine(t, p):
    N, D = t.shape
    NC = p.shape[0]

    def body(t_ref, p_ref, o_ref):
        o_ref[...] = t_ref[...] + jnp.sum(p_ref[...], axis=0)

    return pl.pallas_call(
        body,
        grid=(N // _BR,),
        in_specs=[
            pl.BlockSpec((_BR, D), lambda i: (i, 0)),
            pl.BlockSpec((NC, _BR, D), lambda i: (0, i, 0)),
        ],
        out_specs=pl.BlockSpec((_BR, D), lambda i: (i, 0)),
        out_shape=jax.ShapeDtypeStruct((N, D), jnp.float32),
    )(t, p)


def _sc_aggregate(t, rc3, zeros):
    N, D = t.shape
    _, NW, nch, CH = rc3.shape
    info = plsc.get_sparse_core_info()
    NC, NS = info.num_cores, info.num_subcores
    assert NW == NC * NS and N % NS == 0 and D % 16 == 0
    RPT = N // NS
    SB = max(s for s in range(2, 41, 2) if nch % s == 0)
    NSB = nch // SB
    mesh = plsc.VectorSubcoreMesh(core_axis_name="c", subcore_axis_name="s")

    @functools.partial(
        pl.kernel,
        out_type=jax.ShapeDtypeStruct((NC, NS, RPT, D), jnp.float32),
        mesh=mesh,
        scratch_types=[
            pltpu.VMEM((SB, CH), jnp.int32),
            pltpu.VMEM((SB, CH), jnp.int32),
            pltpu.VMEM((CH, D), jnp.float32),
            pltpu.VMEM((CH, D), jnp.float32),
            pltpu.VMEM_SHARED((N, D), jnp.float32),
            pltpu.SemaphoreType.DMA,
            pltpu.SemaphoreType.DMA,
        ],
    )
    def k(t_hbm, rc_hbm, z_hbm, out_hbm, rowb, colb, r0, r1, acc, s0, s1):
        cid = lax.axis_index("c")
        sid = lax.axis_index("s")
        wid = sid * NC + cid
        rows = (r0, r1)
        sems = (s0, s1)

        pltpu.sync_copy(z_hbm, acc.at[pl.ds(sid * RPT, RPT)])
        plsc.subcore_barrier()

        def fire(j, b):
            pltpu.async_copy(t_hbm.at[rowb.at[j]], rows[b], sems[b])

        def wait(j, b):
            pltpu.make_async_copy(t_hbm.at[rowb.at[j]], rows[b], sems[b]).wait()

        def scat(j, b):
            pltpu.sync_copy(rows[b], acc.at[colb.at[j]], add=True)

        def sblock(s, c):
            pltpu.sync_copy(rc_hbm.at[0, wid, pl.ds(s * SB, SB)], rowb)
            pltpu.sync_copy(rc_hbm.at[1, wid, pl.ds(s * SB, SB)], colb)
            fire(0, 0)
            fire(1, 1)

            def body(i, c2):
                j = 2 * i
                for b in range(2):
                    wait(j + b, b)
                    scat(j + b, b)
                    fire(j + b + 2, b)
                return c2

            lax.fori_loop(0, SB // 2 - 1, body, 0)
            for b in range(2):
                wait(SB - 2 + b, b)
                scat(SB - 2 + b, b)
            return c

        lax.fori_loop(0, NSB, sblock, 0)
        plsc.subcore_barrier()
        pltpu.sync_copy(acc.at[pl.ds(sid * RPT, RPT)], out_hbm.at[cid, sid])

    return k(t, rc3, zeros).reshape(NC, N, D)


def kernel(node_features, edge_index, weight0, bias0, weight1, bias1, hidden_dim):
    del hidden_dim
    N = node_features.shape[0]
    E = edge_index.shape[1]
    info = plsc.get_sparse_core_info()
    NW = info.num_cores * info.num_subcores
    EPW = E // NW
    assert E % NW == 0
    CH = max(c for c in range(1, 129) if EPW % c == 0 and (EPW // c) % 2 == 0)
    rc3 = edge_index.reshape(2, NW, EPW // CH, CH)
    NS = info.num_subcores
    zeros = jnp.zeros((N // NS, node_features.shape[1]), jnp.float32)
    t0 = _dense(node_features, weight0[0], bias0)
    p0 = _sc_aggregate(t0, rc3, zeros)
    t1 = _combine_relu_dense(t0, p0, weight1[0], bias1)
    p1 = _sc_aggregate(t1, rc3, zeros)
    return _combine(t1, p1)
